# baseline (speedup 1.0000x reference)
import jax
import jax.numpy as jnp
from jax import lax
from jax.experimental import pallas as pl
from jax.experimental.pallas import tpu as pltpu
from jax.experimental.pallas import tpu_sc as plsc

TLEN, BATCH, DH, SRC, TGT, CPN = 128, 8, 512, 512, 8000, 4000
ROWS = TLEN * BATCH
ACC = 4016
NWORK = 32
WPB = NWORK // BATCH
TPW = TLEN // WPB
NIDX = SRC // 128
TBLK = 4
NSP = TPW // TBLK
AWORDS = TBLK * ACC


def _sc_body(scores_hbm, idx_hbm, ssum_hbm, idx_v, vals_v, zbuf_v, obuf_v,
             acc_sh, semz, semv, sems, semo):
    cid = lax.axis_index("c")
    sid = lax.axis_index("s")
    w = sid * 2 + cid
    b = w // WPB
    t_base = (w % WPB) * TPW
    base = pl.multiple_of(sid * AWORDS, 8)
    pltpu.sync_copy(idx_hbm.at[w], idx_v)

    def zz(k, c):
        zbuf_v[pl.ds(pl.multiple_of(k * 16, 16), 16)] = jnp.zeros(
            (16,), jnp.float32)
        return c

    lax.fori_loop(0, AWORDS // 16, zz, 0, unroll=8)

    def sp(i, c):
        t = t_base + i * TBLK
        cpz = pltpu.async_copy(
            zbuf_v, acc_sh.at[pl.ds(base, AWORDS)], semz)
        cpv = [
            pltpu.async_copy(
                scores_hbm.at[(t + k) * BATCH + b], vals_v.at[k], semv)
            for k in range(TBLK)
        ]
        cpz.wait()
        for cp in cpv:
            cp.wait()
        cps = [
            pltpu.async_copy(
                vals_v.at[j // NIDX, pl.ds((j % NIDX) * 128, 128)],
                acc_sh.at[idx_v.at[j]],
                sems,
                add=True,
            )
            for j in range(TBLK * NIDX)
        ]
        for cp in cps:
            cp.wait()
        stages = [
            pltpu.async_copy(
                acc_sh.at[pl.ds(pl.multiple_of(base + k * ACC, 8), CPN)],
                obuf_v.at[pl.ds(k * CPN, CPN)],
                semz,
            )
            for k in range(TBLK)
        ]
        for cp in stages:
            cp.wait()
        outs = [
            pltpu.async_copy(
                obuf_v.at[pl.ds(k * CPN, CPN)],
                ssum_hbm.at[pl.ds(
                    pl.multiple_of(((t + k) * BATCH + b) * CPN, 8), CPN)],
                semo,
            )
            for k in range(TBLK)
        ]
        for cp in outs:
            cp.wait()
        return c

    lax.fori_loop(0, NSP, sp, 0)


_sc_scatter = pl.kernel(
    _sc_body,
    out_type=jax.ShapeDtypeStruct((ROWS * CPN,), jnp.float32),
    mesh=plsc.VectorSubcoreMesh(core_axis_name="c", subcore_axis_name="s",
                                num_cores=2, num_subcores=16),
    scratch_types=[
        pltpu.VMEM((TBLK * NIDX, 128), jnp.int32),
        pltpu.VMEM((TBLK, SRC), jnp.float32),
        pltpu.VMEM((AWORDS,), jnp.float32),
        pltpu.VMEM((TBLK * CPN,), jnp.float32),
        pltpu.VMEM_SHARED((16 * AWORDS,), jnp.float32),
        pltpu.SemaphoreType.DMA,
        pltpu.SemaphoreType.DMA,
        pltpu.SemaphoreType.DMA,
        pltpu.SemaphoreType.DMA,
    ],
)


def _tc_body(outh_ref, rawh_ref, wout_ref, bout_ref, wcp_ref, bcp_ref,
             ssum_ref, out_ref, cs_ref):
    x = jnp.dot(outh_ref[...], wout_ref[...],
                preferred_element_type=jnp.float32) + bout_ref[...]
    m = jnp.max(x, axis=1, keepdims=True)
    lse = jnp.log(jnp.sum(jnp.exp(x - m), axis=1, keepdims=True)) + m
    out_ref[:, 0:TGT] = x - lse
    r = jnp.sum(rawh_ref[...] * wcp_ref[...], axis=1, keepdims=True)
    cs = jax.nn.sigmoid(r + bcp_ref[...])
    cs_ref[...] = cs
    out_ref[:, TGT:] = jnp.log(jnp.clip(ssum_ref[...] * cs, 1e-6, 1.0 - 1e-6))


TBR = 128


def _tc_call(outh2, raw2, W_out, bout2, wcp2, bcp2, ssum):
    return pl.pallas_call(
        _tc_body,
        grid=(ROWS // TBR,),
        in_specs=[
            pl.BlockSpec((TBR, DH), lambda i: (i, 0)),
            pl.BlockSpec((TBR, DH), lambda i: (i, 0)),
            pl.BlockSpec((DH, TGT), lambda i: (0, 0)),
            pl.BlockSpec((1, TGT), lambda i: (0, 0)),
            pl.BlockSpec((1, DH), lambda i: (0, 0)),
            pl.BlockSpec((1, 1), lambda i: (0, 0)),
            pl.BlockSpec((TBR, CPN), lambda i: (i, 0)),
        ],
        out_specs=[
            pl.BlockSpec((TBR, TGT + CPN), lambda i: (i, 0)),
            pl.BlockSpec((TBR, 1), lambda i: (i, 0)),
        ],
        out_shape=[
            jax.ShapeDtypeStruct((ROWS, TGT + CPN), jnp.float32),
            jax.ShapeDtypeStruct((ROWS, 1), jnp.float32),
        ],
    )(outh2, raw2, W_out, bout2, wcp2, bcp2, ssum)


def kernel(raw_decoder_hidden, out_decoder_hidden, content, scores,
           copy_to_ext, fields, W_out, b_out, W_cp, b_cp):
    del content
    cte = copy_to_ext.astype(jnp.int32)
    masked = jnp.where(cte == fields, CPN, cte).T
    wids = jnp.arange(NWORK)
    per_w = masked[wids // WPB]
    abs_idx = (per_w[:, None, :]
               + ((wids // 2) * AWORDS)[:, None, None]
               + (jnp.arange(TBLK) * ACC)[None, :, None])
    idx_t = abs_idx.reshape(NWORK, TBLK * NIDX, 128).astype(jnp.int32)
    scores2 = scores.reshape(ROWS, SRC)
    ssum = _sc_scatter(scores2, idx_t).reshape(ROWS, CPN)

    outh2 = out_decoder_hidden.reshape(ROWS, DH)
    raw2 = raw_decoder_hidden.reshape(ROWS, DH)
    bout2 = b_out.reshape(1, TGT)
    wcp2 = W_cp.reshape(1, DH)
    bcp2 = b_cp.reshape(1, 1)
    all_lp, cs = _tc_call(outh2, raw2, W_out, bout2, wcp2, bcp2, ssum)
    return (all_lp.reshape(TLEN, BATCH, TGT + CPN),
            cs.reshape(TLEN, BATCH, 1))


# speedup vs baseline: 1.3340x; 1.3340x over previous
import jax
import jax.numpy as jnp
from jax import lax
from jax.experimental import pallas as pl
from jax.experimental.pallas import tpu as pltpu
from jax.experimental.pallas import tpu_sc as plsc

TLEN, BATCH, DH, SRC, TGT, CPN = 128, 8, 512, 512, 8000, 4000
NCORE = 2
NSUB = 16
BPC = BATCH // NCORE
SPB = NSUB // BPC
SCH = SRC // SPB
ACC = 544
ZR = ACC // SPB
ZROW = 512
FR = CPN // 4
GCH = (128, 128, 128, 128, 128, 128, 128, 104)


def _sc_body(scores_hbm, idx_hbm, ssum_hbm, idx_v, vals_v, zbuf_v, obuf_v,
             acc_sh, semz, semv, sems, semo):
    cid = lax.axis_index("c")
    sid = lax.axis_index("s")
    w = cid * NSUB + sid
    cb = sid // SPB
    q = sid % SPB
    b = cid * BPC + cb

    pltpu.sync_copy(idx_hbm.at[w], idx_v)

    def zz(i, c):
        zbuf_v[i // 8, pl.ds(pl.multiple_of((i % 8) * 16, 16), 16)] = (
            jnp.zeros((16,), jnp.float32))
        return c

    lax.fori_loop(0, ZR * 8, zz, 0, unroll=8)

    cpz = pltpu.async_copy(
        zbuf_v, acc_sh.at[pl.ds(cb * ACC + q * ZR, ZR)], semz)
    cpv = pltpu.async_copy(
        scores_hbm.at[b, pl.ds(q * SCH, SCH)], vals_v, semv)
    cpz.wait()
    cpv.wait()
    plsc.subcore_barrier()

    pltpu.async_copy(vals_v, acc_sh.at[idx_v.at[0]], sems, add=True).wait()
    plsc.subcore_barrier()

    fbase = cid * BPC * CPN + sid * FR
    for k, n in enumerate(GCH):
        pltpu.async_copy(acc_sh.at[idx_v.at[1 + k]], obuf_v, semo).wait()
        pltpu.async_copy(
            obuf_v.at[pl.ds(0, n)],
            ssum_hbm.at[pl.ds(fbase + k * 128, n)],
            semo,
        ).wait()


_sc_scatter = pl.kernel(
    _sc_body,
    out_type=jax.ShapeDtypeStruct((BATCH * CPN, 128), jnp.float32),
    mesh=plsc.VectorSubcoreMesh(core_axis_name="c", subcore_axis_name="s",
                                num_cores=NCORE, num_subcores=NSUB),
    scratch_types=[
        pltpu.VMEM((1 + len(GCH), SCH), jnp.int32),
        pltpu.VMEM((SCH, 128), jnp.float32),
        pltpu.VMEM((ZR, 128), jnp.float32),
        pltpu.VMEM((SCH, 128), jnp.float32),
        pltpu.VMEM_SHARED((BPC * ACC, 128), jnp.float32),
        pltpu.SemaphoreType.DMA,
        pltpu.SemaphoreType.DMA,
        pltpu.SemaphoreType.DMA,
        pltpu.SemaphoreType.DMA,
    ],
)


BB = 2


def _tc_body(wout_ref, outh_ref, rawh_ref, bout_ref, wcp_ref, bcp_ref,
             ssum_ref, out_ref, cs_ref):
    h2 = outh_ref[...].reshape(BB * TLEN, DH)
    ht = jnp.transpose(h2).astype(jnp.bfloat16)
    xt = lax.dot_general(
        wout_ref[...], ht, (((1,), (0,)), ((), ())),
        preferred_element_type=jnp.float32) + bout_ref[...]
    m = jnp.max(xt, axis=0, keepdims=True)
    lse = jnp.log(jnp.sum(jnp.exp(xt - m), axis=0, keepdims=True)) + m
    gen = xt - lse
    r = lax.dot_general(
        wcp_ref[...], rawh_ref[...].reshape(BB * TLEN, DH),
        (((1,), (1,)), ((), ())), preferred_element_type=jnp.float32)
    cs = jax.nn.sigmoid(r + bcp_ref[...])
    for k in range(BB):
        out_ref[k, :TGT, :] = gen[:, k * TLEN:(k + 1) * TLEN]
        csk = cs[:, k * TLEN:(k + 1) * TLEN]
        cs_ref[k, 0, :] = csk[0]
        st = ssum_ref[k * CPN:(k + 1) * CPN, :]
        out_ref[k, TGT:, :] = jnp.log(jnp.clip(st * csk, 1e-6, 1.0 - 1e-6))


def _tc_call(W_out, outh3, raw3, bout2, wcp2, bcp2, ssum):
    return pl.pallas_call(
        _tc_body,
        grid=(BATCH // BB,),
        in_specs=[
            pl.BlockSpec((TGT, DH), lambda i: (0, 0)),
            pl.BlockSpec((BB, TLEN, DH), lambda i: (i, 0, 0)),
            pl.BlockSpec((BB, TLEN, DH), lambda i: (i, 0, 0)),
            pl.BlockSpec((TGT, 1), lambda i: (0, 0)),
            pl.BlockSpec((1, DH), lambda i: (0, 0)),
            pl.BlockSpec((1, 1), lambda i: (0, 0)),
            pl.BlockSpec((BB * CPN, TLEN), lambda i: (i, 0)),
        ],
        out_specs=[
            pl.BlockSpec((BB, TGT + CPN, TLEN), lambda i: (i, 0, 0)),
            pl.BlockSpec((BB, 1, TLEN), lambda i: (i, 0, 0)),
        ],
        out_shape=[
            jax.ShapeDtypeStruct((BATCH, TGT + CPN, TLEN), jnp.float32),
            jax.ShapeDtypeStruct((BATCH, 1, TLEN), jnp.float32),
        ],
    )(W_out, outh3, raw3, bout2, wcp2, bcp2, ssum)


def kernel(raw_decoder_hidden, out_decoder_hidden, content, scores,
           copy_to_ext, fields, W_out, b_out, W_cp, b_cp):
    del content
    cte = copy_to_ext.astype(jnp.int32)
    masked = jnp.where(cte == fields, CPN, cte).T
    sm = jnp.sort(masked, axis=1)
    order = jnp.argsort(masked, axis=1)
    prev = jnp.concatenate(
        [jnp.full((BATCH, 1), -1, jnp.int32), sm[:, :-1]], axis=1)
    rank_sorted = jnp.cumsum((sm != prev).astype(jnp.int32), axis=1) - 1
    inv = jnp.argsort(order, axis=1)
    cmap = jnp.take_along_axis(rank_sorted, inv, axis=1)
    brow = jnp.arange(BATCH, dtype=jnp.int32)[:, None]
    gmap_full = jnp.full((BATCH, CPN + 8), ZROW, jnp.int32)
    idx_safe = jnp.where(sm < CPN, sm, CPN)
    gmap = gmap_full.at[brow, idx_safe].set(rank_sorted)[:, :CPN]
    gabs = gmap + (jnp.arange(BATCH, dtype=jnp.int32)[:, None] % BPC) * ACC
    cmap4 = cmap.reshape(NCORE, BPC, SPB, SCH) + (
        jnp.arange(BPC, dtype=jnp.int32) * ACC)[None, :, None, None]
    gw = gabs.reshape(NCORE, NSUB, FR)
    padv = ((jnp.arange(NSUB, dtype=jnp.int32) // SPB) * ACC + ZROW)
    gpad = jnp.broadcast_to(padv[None, :, None],
                            (NCORE, NSUB, len(GCH) * SCH - FR))
    gchunks = jnp.concatenate([gw, gpad], axis=2).reshape(
        NCORE * NSUB, len(GCH), SCH)
    idx_t = jnp.concatenate(
        [cmap4.reshape(NCORE * NSUB, 1, SCH), gchunks],
        axis=1).astype(jnp.int32)

    scores_t = jnp.transpose(scores, (1, 2, 0))
    ssum = _sc_scatter(scores_t, idx_t)

    outh3 = jnp.transpose(out_decoder_hidden, (1, 0, 2))
    raw3 = jnp.transpose(raw_decoder_hidden, (1, 0, 2))
    woutT = jnp.transpose(W_out).astype(jnp.bfloat16)

    bout2 = b_out.reshape(TGT, 1)
    wcp2 = W_cp.reshape(1, DH)
    bcp2 = b_cp.reshape(1, 1)
    out_t, cs_t = _tc_call(woutT, outh3, raw3, bout2, wcp2, bcp2, ssum)
    all_lp = jnp.transpose(out_t, (2, 0, 1))
    cs = jnp.transpose(cs_t, (2, 0, 1))
    return (all_lp, cs)


# speedup vs baseline: 1.3683x; 1.0256x over previous
import jax
import jax.numpy as jnp
from jax import lax
from jax.experimental import pallas as pl
from jax.experimental.pallas import tpu as pltpu
from jax.experimental.pallas import tpu_sc as plsc

TLEN, BATCH, DH, SRC, TGT, CPN = 128, 8, 512, 512, 8000, 4000
NCORE = 2
NSUB = 16
BPC = BATCH // NCORE
SPB = NSUB // BPC
SCH = SRC // SPB
ACC = 544
ZR = ACC // SPB
ZROW = 512
DUMPR = 520
FR = CPN // 4
GCH = (128, 128, 128, 128, 128, 128, 128, 104)


def _sc_body(scores_hbm, idx_hbm, ssum_hbm, idx_v, vals_v, zbuf_v, obuf_v,
             obuf2_v, acc_sh, semz, semv, sems, semo):
    cid = lax.axis_index("c")
    sid = lax.axis_index("s")
    w = cid * NSUB + sid
    cb = sid // SPB
    q = sid % SPB
    b = cid * BPC + cb

    pltpu.sync_copy(idx_hbm.at[w], idx_v)

    def zz(i, c):
        zbuf_v[i // 8, pl.ds(pl.multiple_of((i % 8) * 16, 16), 16)] = (
            jnp.zeros((16,), jnp.float32))
        return c

    lax.fori_loop(0, ZR * 8, zz, 0, unroll=8)

    cpz = pltpu.async_copy(
        zbuf_v, acc_sh.at[pl.ds(cb * ACC + q * ZR, ZR)], semz)
    cpv = pltpu.async_copy(
        scores_hbm.at[b, pl.ds(q * SCH, SCH)], vals_v, semv)
    cpz.wait()
    cpv.wait()
    plsc.subcore_barrier()

    pltpu.async_copy(vals_v, acc_sh.at[idx_v.at[0]], sems, add=True).wait()
    plsc.subcore_barrier()

    fbase = cid * BPC * CPN + sid * FR
    obufs = (obuf_v, obuf2_v)
    nch = len(GCH)
    gcs = [None] * nch
    wcs = [None] * nch
    gcs[0] = pltpu.async_copy(acc_sh.at[idx_v.at[1]], obufs[0], semo)
    for k, n in enumerate(GCH):
        gcs[k].wait()
        wcs[k] = pltpu.async_copy(
            obufs[k % 2].at[pl.ds(0, n)],
            ssum_hbm.at[pl.ds(fbase + k * 128, n)],
            semz,
        )
        if k + 1 < nch:
            if k >= 1:
                wcs[k - 1].wait()
            gcs[k + 1] = pltpu.async_copy(
                acc_sh.at[idx_v.at[2 + k]], obufs[(k + 1) % 2], semo)
    wcs[nch - 2].wait()
    wcs[nch - 1].wait()


_sc_scatter = pl.kernel(
    _sc_body,
    out_type=jax.ShapeDtypeStruct((BATCH * CPN, 128), jnp.float32),
    mesh=plsc.VectorSubcoreMesh(core_axis_name="c", subcore_axis_name="s",
                                num_cores=NCORE, num_subcores=NSUB),
    scratch_types=[
        pltpu.VMEM((1 + len(GCH), SCH), jnp.int32),
        pltpu.VMEM((SCH, 128), jnp.float32),
        pltpu.VMEM((ZR, 128), jnp.float32),
        pltpu.VMEM((SCH, 128), jnp.float32),
        pltpu.VMEM((SCH, 128), jnp.float32),
        pltpu.VMEM_SHARED((BPC * ACC, 128), jnp.float32),
        pltpu.SemaphoreType.DMA,
        pltpu.SemaphoreType.DMA,
        pltpu.SemaphoreType.DMA,
        pltpu.SemaphoreType.DMA,
    ],
)


BB = 2


def _tc_gen_body(wout_ref, outh_ref, rawh_ref, bout_ref, wcp_ref, bcp_ref,
                 out_ref, cs_ref):
    ht = jnp.concatenate(
        [jnp.transpose(outh_ref[:, k * DH:(k + 1) * DH]) for k in range(BB)],
        axis=1).astype(jnp.bfloat16)
    xt = lax.dot_general(
        wout_ref[...].astype(jnp.bfloat16), ht, (((1,), (0,)), ((), ())),
        preferred_element_type=jnp.float32) + bout_ref[...]
    m = jnp.max(xt, axis=0, keepdims=True)
    lse = jnp.log(jnp.sum(jnp.exp(xt - m), axis=0, keepdims=True)) + m
    gen = xt - lse
    r = jnp.concatenate(
        [lax.dot_general(wcp_ref[...], rawh_ref[:, k * DH:(k + 1) * DH],
                         (((1,), (1,)), ((), ())),
                         preferred_element_type=jnp.float32)
         for k in range(BB)], axis=1)
    cs = jax.nn.sigmoid(r + bcp_ref[...])
    for k in range(BB):
        out_ref[k, :, :] = gen[:, k * TLEN:(k + 1) * TLEN]
        cs_ref[k, 0, :] = cs[0, k * TLEN:(k + 1) * TLEN]


def _tc_gen_call(woutT, outh3, raw3, bout2, wcp2, bcp2):
    return pl.pallas_call(
        _tc_gen_body,
        grid=(BATCH // BB,),
        in_specs=[
            pl.BlockSpec((TGT, DH), lambda i: (0, 0)),
            pl.BlockSpec((TLEN, BB * DH), lambda i: (0, i)),
            pl.BlockSpec((TLEN, BB * DH), lambda i: (0, i)),
            pl.BlockSpec((TGT, 1), lambda i: (0, 0)),
            pl.BlockSpec((1, DH), lambda i: (0, 0)),
            pl.BlockSpec((1, 1), lambda i: (0, 0)),
        ],
        out_specs=[
            pl.BlockSpec((BB, TGT, TLEN), lambda i: (i, 0, 0)),
            pl.BlockSpec((BB, 1, TLEN), lambda i: (i, 0, 0)),
        ],
        out_shape=[
            jax.ShapeDtypeStruct((BATCH, TGT + CPN, TLEN), jnp.float32),
            jax.ShapeDtypeStruct((BATCH, 1, TLEN), jnp.float32),
        ],
    )(woutT, outh3, raw3, bout2, wcp2, bcp2)


def _tc_copy_body(buf_ref, cs_ref, ssum_ref, out_ref):
    del buf_ref
    for k in range(BB):
        csk = cs_ref[k]
        st = ssum_ref[k * CPN:(k + 1) * CPN, :]
        out_ref[k, :, :] = jnp.log(jnp.clip(st * csk, 1e-6, 1.0 - 1e-6))


def _tc_copy_call(buf, cs, ssum):
    return pl.pallas_call(
        _tc_copy_body,
        grid=(BATCH // BB,),
        in_specs=[
            pl.BlockSpec(memory_space=pl.ANY),
            pl.BlockSpec((BB, 1, TLEN), lambda i: (i, 0, 0)),
            pl.BlockSpec((BB * CPN, TLEN), lambda i: (i, 0)),
        ],
        out_specs=pl.BlockSpec((BB, CPN, TLEN), lambda i: (i, 2, 0)),
        out_shape=jax.ShapeDtypeStruct((BATCH, TGT + CPN, TLEN), jnp.float32),
        input_output_aliases={0: 0},
    )(buf, cs, ssum)


def kernel(raw_decoder_hidden, out_decoder_hidden, content, scores,
           copy_to_ext, fields, W_out, b_out, W_cp, b_cp):
    del content
    cte = copy_to_ext.astype(jnp.int32)
    masked = jnp.where(cte == fields, CPN, cte).T
    brow = jnp.arange(BATCH, dtype=jnp.int32)[:, None]
    idx_park = jnp.where(masked < CPN, masked, CPN)
    present = jnp.zeros((BATCH, CPN + 8), jnp.int32).at[
        brow, idx_park].set(1, unique_indices=True)[:, :CPN]
    prefix = jnp.cumsum(present, axis=1)
    gidx = jnp.minimum(masked, CPN - 1)
    cmap = jnp.take_along_axis(prefix, gidx, axis=1) - 1
    cmap = jnp.where(masked == CPN, DUMPR, cmap)
    gmap = jnp.where(present == 1, prefix - 1, ZROW)
    gabs = gmap + (jnp.arange(BATCH, dtype=jnp.int32)[:, None] % BPC) * ACC
    cmap4 = cmap.reshape(NCORE, BPC, SPB, SCH) + (
        jnp.arange(BPC, dtype=jnp.int32) * ACC)[None, :, None, None]
    gw = gabs.reshape(NCORE, NSUB, FR)
    padv = ((jnp.arange(NSUB, dtype=jnp.int32) // SPB) * ACC + ZROW)
    gpad = jnp.broadcast_to(padv[None, :, None],
                            (NCORE, NSUB, len(GCH) * SCH - FR))
    gchunks = jnp.concatenate([gw, gpad], axis=2).reshape(
        NCORE * NSUB, len(GCH), SCH)
    idx_t = jnp.concatenate(
        [cmap4.reshape(NCORE * NSUB, 1, SCH), gchunks],
        axis=1).astype(jnp.int32)

    scores_t = jnp.transpose(scores, (1, 2, 0))
    ssum = _sc_scatter(scores_t, idx_t)

    outh3 = out_decoder_hidden.reshape(TLEN, BATCH * DH)
    raw3 = raw_decoder_hidden.reshape(TLEN, BATCH * DH)
    woutT = jnp.transpose(W_out)

    bout2 = b_out.reshape(TGT, 1)
    wcp2 = W_cp.reshape(1, DH)
    bcp2 = b_cp.reshape(1, 1)
    buf, cs_t = _tc_gen_call(woutT, outh3, raw3, bout2, wcp2, bcp2)
    out_t = _tc_copy_call(buf, cs_t, ssum)
    all_lp = jnp.transpose(out_t, (2, 0, 1))
    cs = jnp.transpose(cs_t, (2, 0, 1))
    return (all_lp, cs)


# speedup vs baseline: 1.6326x; 1.1932x over previous
import jax
import jax.numpy as jnp
from jax import lax
from jax.experimental import pallas as pl
from jax.experimental.pallas import tpu as pltpu
from jax.experimental.pallas import tpu_sc as plsc

TLEN, BATCH, DH, SRC, TGT, CPN = 128, 8, 512, 512, 8000, 4000
NCORE = 2
NSUB = 16
BPC = BATCH // NCORE
SPB = NSUB // BPC
SCH = SRC // SPB
ACC = 544
ZR = ACC // SPB
ZROW = 512
DUMPR = 520
FR = CPN // 4
GCH = (128, 128, 128, 128, 128, 128, 128, 104)


PPAD = 4032


def _sc_plan_body(masked_hbm, cmap_hbm, gmap_hbm, msk_v, pres_v, pref_v,
                  cout_v, gout_v, sem):
    cid = lax.axis_index("c")
    sid = lax.axis_index("s")
    w = cid * NSUB + sid

    @pl.when(w < BATCH)
    def _():
        b = w
        off = (b % BPC) * ACC
        pltpu.sync_copy(masked_hbm.at[pl.ds(
            pl.multiple_of(b * SRC, 8), SRC)], msk_v)

        def z(k, c):
            pres_v[pl.ds(pl.multiple_of(k * 16, 16), 16)] = jnp.zeros(
                (16,), jnp.int32)
            return c

        lax.fori_loop(0, PPAD // 16, z, 0, unroll=8)
        ones = jnp.ones((16,), jnp.int32)

        for k in range(SRC // 16):
            mv = msk_v[pl.ds(k * 16, 16)]
            plsc.store_scatter(pres_v, [mv], ones)

        def cum(k, carry):
            v = pres_v[pl.ds(pl.multiple_of(k * 16, 16), 16)]
            c2 = plsc.cumsum(v) + carry
            pref_v[pl.ds(pl.multiple_of(k * 16, 16), 16)] = c2
            return jnp.max(c2)

        lax.fori_loop(0, PPAD // 16, cum, jnp.int32(0))

        def cm(k, c):
            mv = msk_v[pl.ds(pl.multiple_of(k * 16, 16), 16)]
            mg = jnp.minimum(mv, CPN - 1)
            r = plsc.load_gather(pref_v, [mg]) - 1
            cout_v[pl.ds(pl.multiple_of(k * 16, 16), 16)] = (
                jnp.where(mv == CPN, DUMPR, r) + off)
            return c

        lax.fori_loop(0, SRC // 16, cm, 0, unroll=2)
        pltpu.sync_copy(cout_v, cmap_hbm.at[pl.ds(
            pl.multiple_of(b * SRC, 8), SRC)])

        def gm(k, c):
            sl = pl.ds(pl.multiple_of(k * 16, 16), 16)
            g = jnp.where(pres_v[sl] == 1, pref_v[sl] - 1, ZROW) + off
            gout_v[sl] = g
            return c

        lax.fori_loop(0, PPAD // 16, gm, 0, unroll=8)
        pltpu.sync_copy(gout_v, gmap_hbm.at[pl.ds(
            pl.multiple_of(b * PPAD, 8), PPAD)])


_sc_plan = pl.kernel(
    _sc_plan_body,
    out_type=[
        jax.ShapeDtypeStruct((BATCH * SRC,), jnp.int32),
        jax.ShapeDtypeStruct((BATCH * PPAD,), jnp.int32),
    ],
    mesh=plsc.VectorSubcoreMesh(core_axis_name="c", subcore_axis_name="s",
                                num_cores=NCORE, num_subcores=NSUB),
    scratch_types=[
        pltpu.VMEM((SRC,), jnp.int32),
        pltpu.VMEM((PPAD,), jnp.int32),
        pltpu.VMEM((PPAD,), jnp.int32),
        pltpu.VMEM((SRC,), jnp.int32),
        pltpu.VMEM((PPAD,), jnp.int32),
        pltpu.SemaphoreType.DMA,
    ],
    compiler_params=pltpu.CompilerParams(needs_layout_passes=False),
)


def _sc_body(scores_hbm, cmap_hbm, gmap_hbm, ssum_hbm, idx_v, vals_v,
             zbuf_v, obuf_v, obuf2_v, acc_sh, semz, semv, sems, semo):
    cid = lax.axis_index("c")
    sid = lax.axis_index("s")
    w = cid * NSUB + sid
    cb = sid // SPB
    q = sid % SPB
    b = cid * BPC + cb

    icps = [pltpu.async_copy(
        cmap_hbm.at[pl.ds(pl.multiple_of(b * SRC + q * SCH, 8), SCH)],
        idx_v.at[0], semv)]
    for k in range(len(GCH)):
        icps.append(pltpu.async_copy(
            gmap_hbm.at[pl.ds(
                pl.multiple_of(b * PPAD + q * FR + k * 128, 8), 128)],
            idx_v.at[1 + k], semv))
    for cp in icps:
        cp.wait()

    def zz(i, c):
        zbuf_v[i // 8, pl.ds(pl.multiple_of((i % 8) * 16, 16), 16)] = (
            jnp.zeros((16,), jnp.float32))
        return c

    lax.fori_loop(0, ZR * 8, zz, 0, unroll=8)

    cpz = pltpu.async_copy(
        zbuf_v, acc_sh.at[pl.ds(cb * ACC + q * ZR, ZR)], semz)
    cpv = pltpu.async_copy(
        scores_hbm.at[b, pl.ds(q * SCH, SCH)], vals_v, semv)
    cpz.wait()
    cpv.wait()
    plsc.subcore_barrier()

    pltpu.async_copy(vals_v, acc_sh.at[idx_v.at[0]], sems, add=True).wait()
    plsc.subcore_barrier()

    fbase = cid * BPC * CPN + sid * FR
    obufs = (obuf_v, obuf2_v)
    nch = len(GCH)
    gcs = [None] * nch
    wcs = [None] * nch
    gcs[0] = pltpu.async_copy(acc_sh.at[idx_v.at[1]], obufs[0], semo)
    for k, n in enumerate(GCH):
        gcs[k].wait()
        wcs[k] = pltpu.async_copy(
            obufs[k % 2].at[pl.ds(0, n)],
            ssum_hbm.at[pl.ds(fbase + k * 128, n)],
            semz,
        )
        if k + 1 < nch:
            if k >= 1:
                wcs[k - 1].wait()
            gcs[k + 1] = pltpu.async_copy(
                acc_sh.at[idx_v.at[2 + k]], obufs[(k + 1) % 2], semo)
    wcs[nch - 2].wait()
    wcs[nch - 1].wait()


_sc_scatter = pl.kernel(
    _sc_body,
    out_type=jax.ShapeDtypeStruct((BATCH * CPN, 128), jnp.float32),
    mesh=plsc.VectorSubcoreMesh(core_axis_name="c", subcore_axis_name="s",
                                num_cores=NCORE, num_subcores=NSUB),
    scratch_types=[
        pltpu.VMEM((1 + len(GCH), 128), jnp.int32),
        pltpu.VMEM((SCH, 128), jnp.float32),
        pltpu.VMEM((ZR, 128), jnp.float32),
        pltpu.VMEM((SCH, 128), jnp.float32),
        pltpu.VMEM((SCH, 128), jnp.float32),
        pltpu.VMEM_SHARED((BPC * ACC, 128), jnp.float32),
        pltpu.SemaphoreType.DMA,
        pltpu.SemaphoreType.DMA,
        pltpu.SemaphoreType.DMA,
        pltpu.SemaphoreType.DMA,
    ],
)


BB = 2


def _tc_gen_body(wout_ref, outh_ref, rawh_ref, bout_ref, wcp_ref, bcp_ref,
                 out_ref, cs_ref):
    ht = jnp.concatenate(
        [jnp.transpose(outh_ref[:, k * DH:(k + 1) * DH]) for k in range(BB)],
        axis=1).astype(jnp.bfloat16)
    xt = lax.dot_general(
        wout_ref[...].astype(jnp.bfloat16), ht, (((1,), (0,)), ((), ())),
        preferred_element_type=jnp.float32) + bout_ref[...]
    m = jnp.max(xt, axis=0, keepdims=True)
    lse = jnp.log(jnp.sum(jnp.exp(xt - m), axis=0, keepdims=True)) + m
    gen = xt - lse
    r = jnp.concatenate(
        [lax.dot_general(wcp_ref[...], rawh_ref[:, k * DH:(k + 1) * DH],
                         (((1,), (1,)), ((), ())),
                         preferred_element_type=jnp.float32)
         for k in range(BB)], axis=1)
    cs = jax.nn.sigmoid(r + bcp_ref[...])
    for k in range(BB):
        out_ref[k, :, :] = gen[:, k * TLEN:(k + 1) * TLEN]
        cs_ref[k, 0, :] = cs[0, k * TLEN:(k + 1) * TLEN]


def _tc_gen_call(woutT, outh3, raw3, bout2, wcp2, bcp2):
    return pl.pallas_call(
        _tc_gen_body,
        grid=(BATCH // BB,),
        in_specs=[
            pl.BlockSpec((TGT, DH), lambda i: (0, 0)),
            pl.BlockSpec((TLEN, BB * DH), lambda i: (0, i)),
            pl.BlockSpec((TLEN, BB * DH), lambda i: (0, i)),
            pl.BlockSpec((TGT, 1), lambda i: (0, 0)),
            pl.BlockSpec((1, DH), lambda i: (0, 0)),
            pl.BlockSpec((1, 1), lambda i: (0, 0)),
        ],
        out_specs=[
            pl.BlockSpec((BB, TGT, TLEN), lambda i: (i, 0, 0)),
            pl.BlockSpec((BB, 1, TLEN), lambda i: (i, 0, 0)),
        ],
        out_shape=[
            jax.ShapeDtypeStruct((BATCH, TGT + CPN, TLEN), jnp.float32),
            jax.ShapeDtypeStruct((BATCH, 1, TLEN), jnp.float32),
        ],
    )(woutT, outh3, raw3, bout2, wcp2, bcp2)


def _tc_copy_body(buf_ref, cs_ref, ssum_ref, out_ref):
    del buf_ref
    for k in range(BB):
        csk = cs_ref[k]
        st = ssum_ref[k * CPN:(k + 1) * CPN, :]
        out_ref[k, :, :] = jnp.log(jnp.clip(st * csk, 1e-6, 1.0 - 1e-6))


def _tc_copy_call(buf, cs, ssum):
    return pl.pallas_call(
        _tc_copy_body,
        grid=(BATCH // BB,),
        in_specs=[
            pl.BlockSpec(memory_space=pl.ANY),
            pl.BlockSpec((BB, 1, TLEN), lambda i: (i, 0, 0)),
            pl.BlockSpec((BB * CPN, TLEN), lambda i: (i, 0)),
        ],
        out_specs=pl.BlockSpec((BB, CPN, TLEN), lambda i: (i, 2, 0)),
        out_shape=jax.ShapeDtypeStruct((BATCH, TGT + CPN, TLEN), jnp.float32),
        input_output_aliases={0: 0},
    )(buf, cs, ssum)


def kernel(raw_decoder_hidden, out_decoder_hidden, content, scores,
           copy_to_ext, fields, W_out, b_out, W_cp, b_cp):
    del content
    cte = copy_to_ext.astype(jnp.int32)
    masked = jnp.where(cte == fields, CPN, cte).T.reshape(-1)
    cmap_hbm, gmap_hbm = _sc_plan(masked.astype(jnp.int32))

    scores_t = jnp.transpose(scores, (1, 2, 0))
    ssum = _sc_scatter(scores_t, cmap_hbm, gmap_hbm)

    outh3 = out_decoder_hidden.reshape(TLEN, BATCH * DH)
    raw3 = raw_decoder_hidden.reshape(TLEN, BATCH * DH)
    woutT = jnp.transpose(W_out)

    bout2 = b_out.reshape(TGT, 1)
    wcp2 = W_cp.reshape(1, DH)
    bcp2 = b_cp.reshape(1, 1)
    buf, cs_t = _tc_gen_call(woutT, outh3, raw3, bout2, wcp2, bcp2)
    out_t = _tc_copy_call(buf, cs_t, ssum)
    all_lp = jnp.transpose(out_t, (2, 0, 1))
    cs = jnp.transpose(cs_t, (2, 0, 1))
    return (all_lp, cs)
